# baseline (speedup 1.0000x reference)
import jax
import jax.numpy as jnp
from jax.experimental import pallas as pl

_NUM_CODES = 8192
_DIM = 256
_ROWS = 9216
_R = 256


def _vq_tc_kernel(z_ref, w_ref, znorm_ref, cnorm_ref, zq_ref, counts_ref):
    i = pl.program_id(0)
    z = z_ref[...]
    w = w_ref[...]
    mm = jax.lax.dot_general(
        z, w, (((1,), (1,)), ((), ())),
        preferred_element_type=jnp.float32)
    d = (znorm_ref[...] + cnorm_ref[...]) - 2.0 * mm
    dmin = jnp.min(d, axis=1, keepdims=True)
    col = jax.lax.broadcasted_iota(jnp.int32, d.shape, 1)
    big = jnp.int32(_NUM_CODES)
    ids = jnp.min(jnp.where(d == dmin, col, big), axis=1)
    onehot = (col == ids[:, None]).astype(jnp.float32)
    gathered = jax.lax.dot_general(
        onehot, w, (((1,), (0,)), ((), ())),
        preferred_element_type=jnp.float32,
        precision=jax.lax.Precision.HIGHEST)
    zq_ref[...] = z + (gathered - z)

    @pl.when(i == 0)
    def _():
        counts_ref[...] = jnp.zeros_like(counts_ref)

    counts_ref[...] += jnp.sum(onehot, axis=0, keepdims=True)


def kernel(z, W):
    B, S, D = z.shape
    zf = z.reshape(-1, D)
    znorm = jnp.sum(zf ** 2, axis=1, keepdims=True)
    cnorm = jnp.sum(W ** 2, axis=1)[None, :]

    grid = _ROWS // _R
    zq_flat, counts = pl.pallas_call(
        _vq_tc_kernel,
        grid=(grid,),
        in_specs=[
            pl.BlockSpec((_R, _DIM), lambda i: (i, 0)),
            pl.BlockSpec((_NUM_CODES, _DIM), lambda i: (0, 0)),
            pl.BlockSpec((_R, 1), lambda i: (i, 0)),
            pl.BlockSpec((1, _NUM_CODES), lambda i: (0, 0)),
        ],
        out_specs=[
            pl.BlockSpec((_R, _DIM), lambda i: (i, 0)),
            pl.BlockSpec((1, _NUM_CODES), lambda i: (0, 0)),
        ],
        out_shape=[
            jax.ShapeDtypeStruct((_ROWS, _DIM), jnp.float32),
            jax.ShapeDtypeStruct((1, _NUM_CODES), jnp.float32),
        ],
    )(zf, W, znorm, cnorm)

    avg_probs = counts[0] / (B * S)
    perplexity = jnp.exp(-jnp.sum(avg_probs * jnp.log(avg_probs + 1e-10)))
    return (zq_flat.reshape(z.shape), jnp.asarray(0.0, dtype=jnp.float32),
            perplexity)


# speedup vs baseline: 2.4399x; 2.4399x over previous
import jax
import jax.numpy as jnp
from jax.experimental import pallas as pl
from jax.experimental.pallas import tpu as pltpu
from jax.experimental.pallas import tpu_sc as plsc

_NUM_CODES = 8192
_DIM = 256
_ROWS = 9216
_R = 256
_GW = 128


def _vq_tc_kernel(z_ref, w_ref, znorm_ref, cnorm_ref, ids_ref, counts_ref):
    i = pl.program_id(0)
    z = z_ref[...]
    w = w_ref[...]
    mm = jax.lax.dot_general(
        z, w, (((1,), (1,)), ((), ())),
        preferred_element_type=jnp.float32)
    d = (znorm_ref[...] + cnorm_ref[...]) - 2.0 * mm
    dmin = jnp.min(d, axis=1, keepdims=True)
    col = jax.lax.broadcasted_iota(jnp.int32, d.shape, 1)
    big = jnp.int32(_NUM_CODES)
    ids = jnp.min(jnp.where(d == dmin, col, big), axis=1)
    ids_ref[...] = ids[None, None, :]
    onehot = (col == ids[:, None]).astype(jnp.float32)

    @pl.when(i == 0)
    def _():
        counts_ref[...] = jnp.zeros_like(counts_ref)

    counts_ref[...] += jnp.sum(onehot, axis=0, keepdims=True)


def _sc_gather(W, ids_row):
    mesh = plsc.VectorSubcoreMesh(core_axis_name="core",
                                  subcore_axis_name="subcore")

    @pl.kernel(out_type=jax.ShapeDtypeStruct((_ROWS, _DIM), jnp.float32),
               mesh=mesh)
    def gather_kernel(w_hbm, i_hbm, o_hbm):
        def body(i_vmem, o_vmem):
            pltpu.sync_copy(w_hbm.at[i_vmem.at[0]], o_vmem)

        pltpu.emit_pipeline(
            body,
            grid=(_ROWS // _GW,),
            in_specs=[pl.BlockSpec((1, _GW), index_map=lambda i: (0, i))],
            out_specs=[pl.BlockSpec((_GW, _DIM), index_map=lambda i: (i, 0))],
            core_axis_name=("core", "subcore"),
            dimension_semantics=(pltpu.PARALLEL,),
        )(i_hbm, o_hbm)

    return gather_kernel(W, ids_row)


def kernel(z, W):
    B, S, D = z.shape
    zf = z.reshape(-1, D)
    znorm = jnp.sum(zf ** 2, axis=1, keepdims=True)
    cnorm = jnp.sum(W ** 2, axis=1)[None, :]

    grid = _ROWS // _R
    ids3, counts = pl.pallas_call(
        _vq_tc_kernel,
        grid=(grid,),
        in_specs=[
            pl.BlockSpec((_R, _DIM), lambda i: (i, 0)),
            pl.BlockSpec((_NUM_CODES, _DIM), lambda i: (0, 0)),
            pl.BlockSpec((_R, 1), lambda i: (i, 0)),
            pl.BlockSpec((1, _NUM_CODES), lambda i: (0, 0)),
        ],
        out_specs=[
            pl.BlockSpec((1, 1, _R), lambda i: (i, 0, 0)),
            pl.BlockSpec((1, _NUM_CODES), lambda i: (0, 0)),
        ],
        out_shape=[
            jax.ShapeDtypeStruct((grid, 1, _R), jnp.int32),
            jax.ShapeDtypeStruct((1, _NUM_CODES), jnp.float32),
        ],
    )(zf, W, znorm, cnorm)

    gathered = _sc_gather(W, ids3.reshape(1, _ROWS))
    zq = z + (gathered.reshape(z.shape) - z)

    avg_probs = counts[0] / (B * S)
    perplexity = jnp.exp(-jnp.sum(avg_probs * jnp.log(avg_probs + 1e-10)))
    return (zq, jnp.asarray(0.0, dtype=jnp.float32), perplexity)
